# Initial kernel scaffold; baseline (speedup 1.0000x reference)
import jax
import jax.numpy as jnp
from jax.experimental import pallas as pl


def kernel(x, batch, W1, b1, W2, b2):
    raise NotImplementedError("write your pallas kernel here")


# speedup vs baseline: 4.7274x; 4.7274x over previous
import jax
import jax.numpy as jnp
from jax.experimental import pallas as pl
from jax.experimental.pallas import tpu as pltpu

N = 50000
D = 512
S = 64
B = 1000
NB = N // B

_NEG_INF = float("-inf")


def _sweep_kernel(x_ref, bcol_ref, w1_ref, b1_ref, w2_ref, b2_ref,
                  scores_ref, out_ref, m_out_ref, rd_out_ref,
                  m_ref, d_ref, ot_ref):
    i = pl.program_id(0)

    @pl.when(i == 0)
    def _init():
        m_ref[...] = jnp.full_like(m_ref, _NEG_INF)
        d_ref[...] = jnp.zeros_like(d_ref)
        ot_ref[...] = jnp.zeros_like(ot_ref)

    x = x_ref[...]
    xb = x.astype(jnp.bfloat16)
    h = jnp.tanh(
        jnp.dot(xb, w1_ref[...], preferred_element_type=jnp.float32)
        + b1_ref[...])
    s = (jnp.dot(h.astype(jnp.bfloat16), w2_ref[...],
                 preferred_element_type=jnp.float32)
         + b2_ref[...])
    scores_ref[...] = s

    seg = jax.lax.broadcasted_iota(jnp.int32, (B, S), 1)
    mask = bcol_ref[...] == seg
    maskf = mask.astype(jnp.float32)

    sm = jnp.max(jnp.where(mask, s, _NEG_INF), axis=0, keepdims=True)
    m_old = m_ref[...]
    m_new = jnp.maximum(m_old, sm)
    m_safe = jnp.where(m_new == _NEG_INF, 0.0, m_new)
    r = jnp.where(m_old == _NEG_INF, 0.0, jnp.exp(m_old - m_safe))
    m_ref[...] = m_new

    mg = jnp.dot(maskf, m_safe.reshape(S, 1),
                 preferred_element_type=jnp.float32)
    ex = jnp.exp(s - mg)
    dsum = jax.lax.dot_general(maskf, ex, (((0,), (0,)), ((), ())),
                               preferred_element_type=jnp.float32)
    d_ref[...] = d_ref[...] * r + dsum.reshape(1, S)

    xw = (xb * ex.astype(jnp.bfloat16))
    po = jax.lax.dot_general(xw, mask.astype(jnp.bfloat16),
                             (((0,), (0,)), ((), ())),
                             preferred_element_type=jnp.float32)
    ot_ref[...] = ot_ref[...] * r + po

    @pl.when(i == NB - 1)
    def _fin():
        rd = 1.0 / (d_ref[...] + 1e-16)
        out_ref[...] = (ot_ref[...] * rd).T
        m_out_ref[...] = jnp.where(m_ref[...] == _NEG_INF, 0.0, m_ref[...])
        rd_out_ref[...] = rd


def _weights_kernel(scores_ref, bcol_ref, m_ref, rd_ref, w_ref):
    s = scores_ref[...]
    seg = jax.lax.broadcasted_iota(jnp.int32, (B, S), 1)
    maskf = (bcol_ref[...] == seg).astype(jnp.float32)
    mg = jnp.dot(maskf, m_ref[...].reshape(S, 1),
                 preferred_element_type=jnp.float32)
    rdg = jnp.dot(maskf, rd_ref[...].reshape(S, 1),
                  preferred_element_type=jnp.float32)
    w_ref[...] = jnp.exp(s - mg) * rdg


def kernel(x, batch, W1, b1, W2, b2):
    bcol = batch.astype(jnp.int32).reshape(N, 1)
    w1b = W1.astype(jnp.bfloat16)
    w2b = W2.astype(jnp.bfloat16)
    b1r = b1.reshape(1, D)
    b2r = b2.reshape(1, 1)

    scores, out, m_fin, rd_fin = pl.pallas_call(
        _sweep_kernel,
        grid=(NB,),
        in_specs=[
            pl.BlockSpec((B, D), lambda i: (i, 0)),
            pl.BlockSpec((B, 1), lambda i: (i, 0)),
            pl.BlockSpec((D, D), lambda i: (0, 0)),
            pl.BlockSpec((1, D), lambda i: (0, 0)),
            pl.BlockSpec((D, 1), lambda i: (0, 0)),
            pl.BlockSpec((1, 1), lambda i: (0, 0)),
        ],
        out_specs=[
            pl.BlockSpec((B, 1), lambda i: (i, 0)),
            pl.BlockSpec((S, D), lambda i: (0, 0)),
            pl.BlockSpec((1, S), lambda i: (0, 0)),
            pl.BlockSpec((1, S), lambda i: (0, 0)),
        ],
        out_shape=[
            jax.ShapeDtypeStruct((N, 1), jnp.float32),
            jax.ShapeDtypeStruct((S, D), jnp.float32),
            jax.ShapeDtypeStruct((1, S), jnp.float32),
            jax.ShapeDtypeStruct((1, S), jnp.float32),
        ],
        scratch_shapes=[
            pltpu.VMEM((1, S), jnp.float32),
            pltpu.VMEM((1, S), jnp.float32),
            pltpu.VMEM((D, S), jnp.float32),
        ],
        compiler_params=pltpu.CompilerParams(
            dimension_semantics=("arbitrary",)),
    )(x, bcol, w1b, b1r, w2b, b2r)

    w = pl.pallas_call(
        _weights_kernel,
        grid=(NB,),
        in_specs=[
            pl.BlockSpec((B, 1), lambda i: (i, 0)),
            pl.BlockSpec((B, 1), lambda i: (i, 0)),
            pl.BlockSpec((1, S), lambda i: (0, 0)),
            pl.BlockSpec((1, S), lambda i: (0, 0)),
        ],
        out_specs=pl.BlockSpec((B, 1), lambda i: (i, 0)),
        out_shape=jax.ShapeDtypeStruct((N, 1), jnp.float32),
        compiler_params=pltpu.CompilerParams(
            dimension_semantics=("arbitrary",)),
    )(scores, bcol, m_fin, rd_fin)

    return out, w.reshape(N)


# speedup vs baseline: 6.1062x; 1.2917x over previous
import jax
import jax.numpy as jnp
from jax.experimental import pallas as pl
from jax.experimental.pallas import tpu as pltpu

N = 50000
D = 512
S = 64
B = 1000
NB = N // B
R2 = 5
NB2 = NB // R2

_NEG_INF = float("-inf")


def _sweep_kernel(x_ref, bcol_ref, brow_ref, w1_ref, b1_ref, w2_ref,
                  scores_ref, out_ref, q2_ref,
                  m_ref, d_ref, o_ref):
    i = pl.program_id(0)

    @pl.when(i == 0)
    def _init():
        m_ref[...] = jnp.full_like(m_ref, _NEG_INF)
        d_ref[...] = jnp.zeros_like(d_ref)
        o_ref[...] = jnp.zeros_like(o_ref)

    x = x_ref[...]
    xb = x.astype(jnp.bfloat16)
    h = jnp.tanh(
        jnp.dot(xb, w1_ref[...], preferred_element_type=jnp.float32)
        + b1_ref[...])
    s = jnp.dot(h.astype(jnp.bfloat16), w2_ref[...],
                preferred_element_type=jnp.float32)
    scores_ref[...] = s

    bcol = bcol_ref[...]
    brow = brow_ref[...].reshape(1, B)
    mask = bcol == jax.lax.broadcasted_iota(jnp.int32, (B, S), 1)
    mask_t = brow == jax.lax.broadcasted_iota(jnp.int32, (S, B), 0)
    mask_b = mask.astype(jnp.bfloat16)
    mask_t_b = mask_t.astype(jnp.bfloat16)

    sm = jnp.max(jnp.where(mask, s, _NEG_INF), axis=0, keepdims=True)
    m_old = m_ref[...]
    m_new = jnp.maximum(m_old, sm).astype(jnp.bfloat16).astype(jnp.float32)
    m_safe = jnp.where(m_new == _NEG_INF, 0.0, m_new)
    r = jnp.where(m_old == _NEG_INF, 0.0, jnp.exp(m_old - m_safe))
    m_ref[...] = m_new
    r_col = r.reshape(S, 1)

    mg = jnp.dot(mask_b, m_safe.reshape(S, 1).astype(jnp.bfloat16),
                 preferred_element_type=jnp.float32)
    ex = jnp.exp(s - mg)
    exb = ex.astype(jnp.bfloat16)
    dsum = jnp.dot(mask_t_b, exb,
                   preferred_element_type=jnp.float32)
    d_ref[...] = d_ref[...] * r_col + dsum

    xw = xb * exb
    po = jnp.dot(mask_t_b, xw,
                 preferred_element_type=jnp.float32)
    o_ref[...] = o_ref[...] * r_col + po

    @pl.when(i == NB - 1)
    def _fin():
        d = d_ref[...]
        out_ref[...] = o_ref[...] * (1.0 / (d + 1e-16))
        m_fin = jnp.where(m_ref[...] == _NEG_INF, 0.0, m_ref[...])
        q = m_fin.reshape(S, 1) + jnp.log(d + 1e-16)
        qhi = q.astype(jnp.bfloat16)
        qlo = (q - qhi.astype(jnp.float32)).astype(jnp.bfloat16)
        q2_ref[...] = jnp.concatenate(
            [qhi.reshape(1, S), qlo.reshape(1, S)], axis=0)


def _weights_kernel(scores_ref, brow_ref, q2_ref, w_ref):
    q2 = q2_ref[...]
    for r in range(R2):
        srow = scores_ref[r]
        brow = brow_ref[r]
        mask_t_b = (brow == jax.lax.broadcasted_iota(jnp.int32, (S, B), 0)
                    ).astype(jnp.bfloat16)
        mg2 = jnp.dot(q2, mask_t_b,
                      preferred_element_type=jnp.float32)
        w_ref[r] = jnp.exp(srow - mg2[0:1, :] - mg2[1:2, :])


def kernel(x, batch, W1, b1, W2, b2):
    bi32 = batch.astype(jnp.int32)
    bcol = bi32.reshape(N, 1)
    brow3 = bi32.reshape(NB, 1, B)
    w1b = W1.astype(jnp.bfloat16)
    w2b = W2.astype(jnp.bfloat16)
    b1r = b1.reshape(1, D)

    scores, out, q2 = pl.pallas_call(
        _sweep_kernel,
        grid=(NB,),
        in_specs=[
            pl.BlockSpec((B, D), lambda i: (i, 0)),
            pl.BlockSpec((B, 1), lambda i: (i, 0)),
            pl.BlockSpec((1, 1, B), lambda i: (i, 0, 0)),
            pl.BlockSpec((D, D), lambda i: (0, 0)),
            pl.BlockSpec((1, D), lambda i: (0, 0)),
            pl.BlockSpec((D, 1), lambda i: (0, 0)),
        ],
        out_specs=[
            pl.BlockSpec((B, 1), lambda i: (i, 0)),
            pl.BlockSpec((S, D), lambda i: (0, 0)),
            pl.BlockSpec((2, S), lambda i: (0, 0)),
        ],
        out_shape=[
            jax.ShapeDtypeStruct((N, 1), jnp.float32),
            jax.ShapeDtypeStruct((S, D), jnp.float32),
            jax.ShapeDtypeStruct((2, S), jnp.bfloat16),
        ],
        scratch_shapes=[
            pltpu.VMEM((1, S), jnp.float32),
            pltpu.VMEM((S, 1), jnp.float32),
            pltpu.VMEM((S, D), jnp.float32),
        ],
        compiler_params=pltpu.CompilerParams(
            dimension_semantics=("arbitrary",)),
    )(x, bcol, brow3, w1b, b1r, w2b)

    scores3 = scores.reshape(NB, 1, B)
    w3 = pl.pallas_call(
        _weights_kernel,
        grid=(NB2,),
        in_specs=[
            pl.BlockSpec((R2, 1, B), lambda i: (i, 0, 0)),
            pl.BlockSpec((R2, 1, B), lambda i: (i, 0, 0)),
            pl.BlockSpec((2, S), lambda i: (0, 0)),
        ],
        out_specs=pl.BlockSpec((R2, 1, B), lambda i: (i, 0, 0)),
        out_shape=jax.ShapeDtypeStruct((NB, 1, B), jnp.float32),
        compiler_params=pltpu.CompilerParams(
            dimension_semantics=("arbitrary",)),
    )(scores3, brow3, q2)

    return out, w3.reshape(N)


# speedup vs baseline: 7.3839x; 1.2092x over previous
import jax
import jax.numpy as jnp
from jax.experimental import pallas as pl
from jax.experimental.pallas import tpu as pltpu

N = 50000
D = 512
S = 64
B = 1000
NB = N // B
R2 = 5
NB2 = NB // R2

_NEG_INF = float("-inf")


def _sweep_kernel(x_ref, bcolp_ref, browp_ref, w1_ref, b1_ref,
                  w2_ref,
                  scores_ref, out_ref, q2_ref,
                  m_ref, d_ref, o_ref, sprev_ref, xbprev_ref):
    i = pl.program_id(0)

    @pl.when(i == 0)
    def _init():
        m_ref[...] = jnp.full_like(m_ref, _NEG_INF)
        d_ref[...] = jnp.zeros_like(d_ref)
        o_ref[...] = jnp.zeros_like(o_ref)
        sprev_ref[...] = jnp.full_like(sprev_ref, _NEG_INF)
        xbprev_ref[...] = jnp.zeros_like(xbprev_ref)

    x = x_ref[...]
    xb = x.astype(jnp.bfloat16)
    h = jnp.tanh(
        jnp.dot(xb, w1_ref[...], preferred_element_type=jnp.float32)
        + b1_ref[...])
    s = jnp.dot(h.astype(jnp.bfloat16), w2_ref[...],
                preferred_element_type=jnp.float32)

    s_prev = sprev_ref[...]
    bcolp = bcolp_ref[...]
    browp = browp_ref[...].reshape(1, B)
    mask = bcolp == jax.lax.broadcasted_iota(jnp.int32, (B, S), 1)
    mask_b = mask.astype(jnp.bfloat16)
    mask_t_b = (browp == jax.lax.broadcasted_iota(jnp.int32, (S, B), 0)
                ).astype(jnp.bfloat16)

    sm = jnp.max(jnp.where(mask, s_prev, _NEG_INF), axis=0,
                 keepdims=True)
    m_old = m_ref[...]
    m_new = jnp.maximum(m_old, sm).astype(jnp.bfloat16).astype(jnp.float32)
    m_safe = jnp.where(m_new == _NEG_INF, 0.0, m_new)
    r_col = jnp.where(m_old == _NEG_INF, 0.0,
                      jnp.exp(m_old - m_safe)).reshape(S, 1)
    m_ref[...] = m_new

    mg = jnp.dot(mask_b, m_safe.reshape(S, 1).astype(jnp.bfloat16),
                 preferred_element_type=jnp.float32)
    ex = jnp.exp(s_prev - mg)
    exb = ex.astype(jnp.bfloat16)
    dsum = jnp.dot(mask_t_b, exb,
                   preferred_element_type=jnp.float32)
    d_ref[...] = d_ref[...] * r_col + dsum
    xw = xbprev_ref[...] * exb
    po = jnp.dot(mask_t_b, xw,
                 preferred_element_type=jnp.float32)
    o_ref[...] = o_ref[...] * r_col + po

    scores_ref[...] = s
    sprev_ref[...] = s
    xbprev_ref[...] = xb

    @pl.when(i == NB)
    def _fin():
        d = d_ref[...]
        out_ref[...] = o_ref[...] * (1.0 / (d + 1e-16))
        m_fin = jnp.where(m_ref[...] == _NEG_INF, 0.0, m_ref[...])
        q = m_fin.reshape(S, 1) + jnp.log(d + 1e-16)
        qhi = q.astype(jnp.bfloat16)
        qlo = (q - qhi.astype(jnp.float32)).astype(jnp.bfloat16)
        q2_ref[...] = jnp.concatenate(
            [qhi.reshape(1, S), qlo.reshape(1, S)], axis=0)


def _weights_kernel(scores_ref, brow_ref, q2_ref, w_ref):
    q2 = q2_ref[...]
    for r in range(R2):
        srow = scores_ref[r]
        brow = brow_ref[r]
        mask_t_b = (brow == jax.lax.broadcasted_iota(jnp.int32, (S, B), 0)
                    ).astype(jnp.bfloat16)
        mg2 = jnp.dot(q2, mask_t_b,
                      preferred_element_type=jnp.float32)
        w_ref[r] = jnp.exp(srow - mg2[0:1, :] - mg2[1:2, :])


def kernel(x, batch, W1, b1, W2, b2):
    bi32 = batch.astype(jnp.int32)
    bcol = bi32.reshape(N, 1)
    brow3 = bi32.reshape(NB, 1, B)
    w1b = W1.astype(jnp.bfloat16)
    w2b = W2.astype(jnp.bfloat16)
    b1r = b1.reshape(1, D)

    _clamp = lambda i: jnp.minimum(i, NB - 1)
    _prev = lambda i: jnp.clip(i - 1, 0, NB - 1)
    scores, out, q2 = pl.pallas_call(
        _sweep_kernel,
        grid=(NB + 1,),
        in_specs=[
            pl.BlockSpec((B, D), lambda i: (_clamp(i), 0)),
            pl.BlockSpec((B, 1), lambda i: (_prev(i), 0)),
            pl.BlockSpec((1, 1, B), lambda i: (_prev(i), 0, 0)),
            pl.BlockSpec((D, D), lambda i: (0, 0)),
            pl.BlockSpec((1, D), lambda i: (0, 0)),
            pl.BlockSpec((D, 1), lambda i: (0, 0)),
        ],
        out_specs=[
            pl.BlockSpec((B, 1), lambda i: (_clamp(i), 0)),
            pl.BlockSpec((S, D), lambda i: (0, 0)),
            pl.BlockSpec((2, S), lambda i: (0, 0)),
        ],
        out_shape=[
            jax.ShapeDtypeStruct((N, 1), jnp.float32),
            jax.ShapeDtypeStruct((S, D), jnp.float32),
            jax.ShapeDtypeStruct((2, S), jnp.bfloat16),
        ],
        scratch_shapes=[
            pltpu.VMEM((1, S), jnp.float32),
            pltpu.VMEM((S, 1), jnp.float32),
            pltpu.VMEM((S, D), jnp.float32),
            pltpu.VMEM((B, 1), jnp.float32),
            pltpu.VMEM((B, D), jnp.bfloat16),
        ],
        compiler_params=pltpu.CompilerParams(
            dimension_semantics=("arbitrary",)),
    )(x, bcol, brow3, w1b, b1r, w2b)

    scores3 = scores.reshape(NB, 1, B)
    w3 = pl.pallas_call(
        _weights_kernel,
        grid=(NB2,),
        in_specs=[
            pl.BlockSpec((R2, 1, B), lambda i: (i, 0, 0)),
            pl.BlockSpec((R2, 1, B), lambda i: (i, 0, 0)),
            pl.BlockSpec((2, S), lambda i: (0, 0)),
        ],
        out_specs=pl.BlockSpec((R2, 1, B), lambda i: (i, 0, 0)),
        out_shape=jax.ShapeDtypeStruct((NB, 1, B), jnp.float32),
        compiler_params=pltpu.CompilerParams(
            dimension_semantics=("arbitrary",)),
    )(scores3, brow3, q2)

    return out, w3.reshape(N)


# speedup vs baseline: 9.5612x; 1.2949x over previous
import jax
import jax.numpy as jnp
from jax.experimental import pallas as pl
from jax.experimental.pallas import tpu as pltpu

N = 50000
D = 512
S = 64
B = 1000
NB = N // B
R2 = 5
NB2 = NB // R2

_NEG_INF = float("-inf")


def _sweep_kernel(x_ref, browp_ref, w1_ref, b1_ref,
                  w2_ref,
                  scores_ref, out_ref, q2_ref,
                  m_ref, d_ref, o_ref, sprev_ref, xbprev_ref):
    i = pl.program_id(0)

    @pl.when(i == 0)
    def _init():
        m_ref[...] = jnp.full_like(m_ref, _NEG_INF)
        d_ref[...] = jnp.zeros_like(d_ref)
        o_ref[...] = jnp.zeros_like(o_ref)
        sprev_ref[...] = jnp.full_like(sprev_ref, _NEG_INF)
        xbprev_ref[...] = jnp.zeros_like(xbprev_ref)

    x = x_ref[...]
    xb = x.astype(jnp.bfloat16)
    h = jnp.tanh(
        jnp.dot(xb, w1_ref[...], preferred_element_type=jnp.float32)
        + b1_ref[...])
    s = jnp.dot(h.astype(jnp.bfloat16), w2_ref[...],
                preferred_element_type=jnp.float32)

    s_prev = sprev_ref[...]
    browp = browp_ref[...].reshape(1, B)
    bcolp = browp.reshape(B, 1)
    mask = bcolp == jax.lax.broadcasted_iota(jnp.int32, (B, S), 1)
    mask_b = mask.astype(jnp.bfloat16)
    mask_t_b = (browp == jax.lax.broadcasted_iota(jnp.int32, (S, B), 0)
                ).astype(jnp.bfloat16)

    sm = jnp.max(jnp.where(mask, s_prev, _NEG_INF), axis=0,
                 keepdims=True)
    m_old = m_ref[...]
    m_new = jnp.maximum(m_old, sm).astype(jnp.bfloat16).astype(jnp.float32)
    m_safe = jnp.where(m_new == _NEG_INF, 0.0, m_new)
    r_col = jnp.where(m_old == _NEG_INF, 0.0,
                      jnp.exp(m_old - m_safe)).reshape(S, 1)
    m_ref[...] = m_new

    mg = jnp.dot(mask_b, m_safe.reshape(S, 1).astype(jnp.bfloat16),
                 preferred_element_type=jnp.float32)
    ex = jnp.exp(s_prev - mg)
    exb = ex.astype(jnp.bfloat16)
    dsum = jnp.dot(mask_t_b, exb,
                   preferred_element_type=jnp.float32)
    d_ref[...] = d_ref[...] * r_col + dsum
    xw = xbprev_ref[...] * exb
    po = jnp.dot(mask_t_b, xw,
                 preferred_element_type=jnp.float32)
    o_ref[...] = o_ref[...] * r_col + po

    scores_ref[...] = s.reshape(1, 1, B)
    sprev_ref[...] = s
    xbprev_ref[...] = xb

    @pl.when(i == NB)
    def _fin():
        d = d_ref[...]
        out_ref[...] = o_ref[...] * (1.0 / (d + 1e-16))
        m_fin = jnp.where(m_ref[...] == _NEG_INF, 0.0, m_ref[...])
        q = m_fin.reshape(S, 1) + jnp.log(d + 1e-16)
        qhi = q.astype(jnp.bfloat16)
        qlo = (q - qhi.astype(jnp.float32)).astype(jnp.bfloat16)
        q2_ref[...] = jnp.concatenate(
            [qhi.reshape(1, S), qlo.reshape(1, S)], axis=0)


def _weights_kernel(scores_ref, brow_ref, q2_ref, w_ref):
    q2 = q2_ref[...]
    for r in range(R2):
        srow = scores_ref[r]
        brow = brow_ref[r]
        mask_t_b = (brow == jax.lax.broadcasted_iota(jnp.int32, (S, B), 0)
                    ).astype(jnp.bfloat16)
        mg2 = jnp.dot(q2, mask_t_b,
                      preferred_element_type=jnp.float32)
        w_ref[r] = jnp.exp(srow - mg2[0:1, :] - mg2[1:2, :])


def kernel(x, batch, W1, b1, W2, b2):
    brow3 = batch.astype(jnp.int32).reshape(NB, 1, B)
    w1b = W1.astype(jnp.bfloat16)
    w2b = W2.astype(jnp.bfloat16)
    b1r = b1.reshape(1, D)

    _clamp = lambda i: jnp.minimum(i, NB - 1)
    _prev = lambda i: jnp.clip(i - 1, 0, NB - 1)
    scores, out, q2 = pl.pallas_call(
        _sweep_kernel,
        grid=(NB + 1,),
        in_specs=[
            pl.BlockSpec((B, D), lambda i: (_clamp(i), 0)),
            pl.BlockSpec((1, 1, B), lambda i: (_prev(i), 0, 0)),
            pl.BlockSpec((D, D), lambda i: (0, 0)),
            pl.BlockSpec((1, D), lambda i: (0, 0)),
            pl.BlockSpec((D, 1), lambda i: (0, 0)),
        ],
        out_specs=[
            pl.BlockSpec((1, 1, B), lambda i: (_clamp(i), 0, 0)),
            pl.BlockSpec((S, D), lambda i: (0, 0)),
            pl.BlockSpec((2, S), lambda i: (0, 0)),
        ],
        out_shape=[
            jax.ShapeDtypeStruct((NB, 1, B), jnp.float32),
            jax.ShapeDtypeStruct((S, D), jnp.float32),
            jax.ShapeDtypeStruct((2, S), jnp.bfloat16),
        ],
        scratch_shapes=[
            pltpu.VMEM((1, S), jnp.float32),
            pltpu.VMEM((S, 1), jnp.float32),
            pltpu.VMEM((S, D), jnp.float32),
            pltpu.VMEM((B, 1), jnp.float32),
            pltpu.VMEM((B, D), jnp.bfloat16),
        ],
        compiler_params=pltpu.CompilerParams(
            dimension_semantics=("arbitrary",)),
    )(x, brow3, w1b, b1r, w2b)

    scores3 = scores
    w3 = pl.pallas_call(
        _weights_kernel,
        grid=(NB2,),
        in_specs=[
            pl.BlockSpec((R2, 1, B), lambda i: (i, 0, 0)),
            pl.BlockSpec((R2, 1, B), lambda i: (i, 0, 0)),
            pl.BlockSpec((2, S), lambda i: (0, 0)),
        ],
        out_specs=pl.BlockSpec((R2, 1, B), lambda i: (i, 0, 0)),
        out_shape=jax.ShapeDtypeStruct((NB, 1, B), jnp.float32),
        compiler_params=pltpu.CompilerParams(
            dimension_semantics=("arbitrary",)),
    )(scores3, brow3, q2)

    return out, w3.reshape(N)


# speedup vs baseline: 10.4464x; 1.0926x over previous
import jax
import jax.numpy as jnp
from jax.experimental import pallas as pl
from jax.experimental.pallas import tpu as pltpu

N = 50000
D = 512
S = 64
B = 2000
NB = N // B
R2 = 5
NB2 = NB // R2

_NEG_INF = float("-inf")


def _sweep_kernel(x_ref, browp_ref, w1_ref, b1_ref,
                  w2_ref,
                  scores_ref, out_ref, q2_ref,
                  m_ref, d_ref, o_ref, sprev_ref, xbprev_ref):
    i = pl.program_id(0)

    @pl.when(i == 0)
    def _init():
        m_ref[...] = jnp.full_like(m_ref, _NEG_INF)
        d_ref[...] = jnp.zeros_like(d_ref)
        o_ref[...] = jnp.zeros_like(o_ref)
        sprev_ref[...] = jnp.full_like(sprev_ref, _NEG_INF)
        xbprev_ref[...] = jnp.zeros_like(xbprev_ref)

    x = x_ref[...]
    xb = x.astype(jnp.bfloat16)
    h = jnp.tanh(
        jnp.dot(xb, w1_ref[...], preferred_element_type=jnp.float32)
        + b1_ref[...])
    s = jnp.dot(h.astype(jnp.bfloat16), w2_ref[...],
                preferred_element_type=jnp.float32)

    s_prev = sprev_ref[...]
    browp = browp_ref[...].reshape(1, B)
    bcolp = browp.reshape(B, 1)
    mask = bcolp == jax.lax.broadcasted_iota(jnp.int32, (B, S), 1)
    mask_b = mask.astype(jnp.bfloat16)
    mask_t_b = (browp == jax.lax.broadcasted_iota(jnp.int32, (S, B), 0)
                ).astype(jnp.bfloat16)

    sm = jnp.max(jnp.where(mask, s_prev, _NEG_INF), axis=0,
                 keepdims=True)
    m_old = m_ref[...]
    m_new = jnp.maximum(m_old, sm).astype(jnp.bfloat16).astype(jnp.float32)
    m_safe = jnp.where(m_new == _NEG_INF, 0.0, m_new)
    r_col = jnp.where(m_old == _NEG_INF, 0.0,
                      jnp.exp(m_old - m_safe)).reshape(S, 1)
    m_ref[...] = m_new

    mg = jnp.dot(mask_b, m_safe.reshape(S, 1).astype(jnp.bfloat16),
                 preferred_element_type=jnp.float32)
    ex = jnp.exp(s_prev - mg)
    exb = ex.astype(jnp.bfloat16)
    dsum = jnp.dot(mask_t_b, exb,
                   preferred_element_type=jnp.float32)
    d_ref[...] = d_ref[...] * r_col + dsum
    xw = xbprev_ref[...] * exb
    po = jnp.dot(mask_t_b, xw,
                 preferred_element_type=jnp.float32)
    o_ref[...] = o_ref[...] * r_col + po

    scores_ref[...] = s.reshape(1, 1, B)
    sprev_ref[...] = s
    xbprev_ref[...] = xb

    @pl.when(i == NB)
    def _fin():
        d = d_ref[...]
        out_ref[...] = o_ref[...] * (1.0 / (d + 1e-16))
        m_fin = jnp.where(m_ref[...] == _NEG_INF, 0.0, m_ref[...])
        q = m_fin.reshape(S, 1) + jnp.log(d + 1e-16)
        qhi = q.astype(jnp.bfloat16)
        qlo = (q - qhi.astype(jnp.float32)).astype(jnp.bfloat16)
        q2_ref[...] = jnp.concatenate(
            [qhi.reshape(1, S), qlo.reshape(1, S)], axis=0)


def _weights_kernel(scores_ref, brow_ref, q2_ref, w_ref):
    q2 = q2_ref[...]
    for r in range(R2):
        srow = scores_ref[r]
        brow = brow_ref[r]
        mask_t_b = (brow == jax.lax.broadcasted_iota(jnp.int32, (S, B), 0)
                    ).astype(jnp.bfloat16)
        mg2 = jnp.dot(q2, mask_t_b,
                      preferred_element_type=jnp.float32)
        w_ref[r] = jnp.exp(srow - mg2[0:1, :] - mg2[1:2, :])


def kernel(x, batch, W1, b1, W2, b2):
    brow3 = batch.astype(jnp.int32).reshape(NB, 1, B)
    w1b = W1.astype(jnp.bfloat16)
    w2b = W2.astype(jnp.bfloat16)
    b1r = b1.reshape(1, D)

    _clamp = lambda i: jnp.minimum(i, NB - 1)
    _prev = lambda i: jnp.clip(i - 1, 0, NB - 1)
    scores, out, q2 = pl.pallas_call(
        _sweep_kernel,
        grid=(NB + 1,),
        in_specs=[
            pl.BlockSpec((B, D), lambda i: (_clamp(i), 0)),
            pl.BlockSpec((1, 1, B), lambda i: (_prev(i), 0, 0)),
            pl.BlockSpec((D, D), lambda i: (0, 0)),
            pl.BlockSpec((1, D), lambda i: (0, 0)),
            pl.BlockSpec((D, 1), lambda i: (0, 0)),
        ],
        out_specs=[
            pl.BlockSpec((1, 1, B), lambda i: (_clamp(i), 0, 0)),
            pl.BlockSpec((S, D), lambda i: (0, 0)),
            pl.BlockSpec((2, S), lambda i: (0, 0)),
        ],
        out_shape=[
            jax.ShapeDtypeStruct((NB, 1, B), jnp.float32),
            jax.ShapeDtypeStruct((S, D), jnp.float32),
            jax.ShapeDtypeStruct((2, S), jnp.bfloat16),
        ],
        scratch_shapes=[
            pltpu.VMEM((1, S), jnp.float32),
            pltpu.VMEM((S, 1), jnp.float32),
            pltpu.VMEM((S, D), jnp.float32),
            pltpu.VMEM((B, 1), jnp.float32),
            pltpu.VMEM((B, D), jnp.bfloat16),
        ],
        compiler_params=pltpu.CompilerParams(
            dimension_semantics=("arbitrary",)),
    )(x, brow3, w1b, b1r, w2b)

    scores3 = scores
    w3 = pl.pallas_call(
        _weights_kernel,
        grid=(NB2,),
        in_specs=[
            pl.BlockSpec((R2, 1, B), lambda i: (i, 0, 0)),
            pl.BlockSpec((R2, 1, B), lambda i: (i, 0, 0)),
            pl.BlockSpec((2, S), lambda i: (0, 0)),
        ],
        out_specs=pl.BlockSpec((R2, 1, B), lambda i: (i, 0, 0)),
        out_shape=jax.ShapeDtypeStruct((NB, 1, B), jnp.float32),
        compiler_params=pltpu.CompilerParams(
            dimension_semantics=("arbitrary",)),
    )(scores3, brow3, q2)

    return out, w3.reshape(N)
